# baseline (speedup 1.0000x reference)
import functools

import jax
import jax.numpy as jnp
from jax import lax
from jax.experimental import pallas as pl
from jax.experimental.pallas import tpu as pltpu
from jax.experimental.pallas import tpu_sc as plsc

BATCH = 16384
DIM = 16
NC = 2
NS = 16
NW = NC * NS
B_PER_W = BATCH // NW
CHUNK = 128
NCHUNK = B_PER_W // CHUNK

_mesh = plsc.VectorSubcoreMesh(
    core_axis_name="c", subcore_axis_name="s", num_cores=NC, num_subcores=NS)


@functools.partial(
    pl.kernel,
    mesh=_mesh,
    out_type=(
        jax.ShapeDtypeStruct((BATCH, DIM), jnp.float32),
        jax.ShapeDtypeStruct((BATCH, DIM), jnp.float32),
    ),
    scratch_types=(
        pltpu.VMEM((B_PER_W,), jnp.int32),
        pltpu.VMEM((B_PER_W,), jnp.int32),
        pltpu.VMEM((B_PER_W,), jnp.int32),
        pltpu.VMEM((B_PER_W, DIM), jnp.float32),
        pltpu.VMEM((B_PER_W, DIM), jnp.float32),
        pltpu.VMEM((B_PER_W, DIM), jnp.float32),
        pltpu.SemaphoreType.DMA,
    ),
    compiler_params=pltpu.CompilerParams(use_tc_tiling_on_sc=False),
)
def _bdl_fwd(user_hbm, item_hbm, neg_hbm, uw_hbm, iw_hbm,
             out_ui, out_uj, idx_u, idx_i, idx_j, ru, ri, rj, sem):
    wid = lax.axis_index("s") * NC + lax.axis_index("c")
    base = wid * B_PER_W

    pltpu.sync_copy(user_hbm.at[pl.ds(base, B_PER_W)], idx_u)
    pltpu.sync_copy(item_hbm.at[pl.ds(base, B_PER_W)], idx_i)
    pltpu.sync_copy(neg_hbm.at[pl.ds(base, B_PER_W)], idx_j)

    copies = []
    for c in range(NCHUNK):
        sl = pl.ds(c * CHUNK, CHUNK)
        copies.append(pltpu.async_copy(uw_hbm.at[idx_u.at[sl]], ru.at[sl], sem))
        copies.append(pltpu.async_copy(iw_hbm.at[idx_i.at[sl]], ri.at[sl], sem))
        copies.append(pltpu.async_copy(iw_hbm.at[idx_j.at[sl]], rj.at[sl], sem))
    for cp in copies:
        cp.wait()

    def body(r, _):
        u = ru[r, :]
        ri[r, :] = u * ri[r, :]
        rj[r, :] = u * rj[r, :]
        return 0

    lax.fori_loop(0, B_PER_W, body, 0)

    pltpu.sync_copy(ri, out_ui.at[pl.ds(base, B_PER_W)])
    pltpu.sync_copy(rj, out_uj.at[pl.ds(base, B_PER_W)])


def kernel(user, item, neg_item, user_emb_w, item_emb_w):
    return _bdl_fwd(user.astype(jnp.int32), item.astype(jnp.int32),
                    neg_item.astype(jnp.int32), user_emb_w, item_emb_w)


# speedup vs baseline: 4.6475x; 4.6475x over previous
import functools

import jax
import jax.numpy as jnp
from jax import lax
from jax.experimental import pallas as pl
from jax.experimental.pallas import tpu as pltpu
from jax.experimental.pallas import tpu_sc as plsc

V = 1000000
BATCH = 16384
DIM = 16
NC = 2
NS = 16
NW = NC * NS
B_PER_W = BATCH // NW
G = 16
NG = B_PER_W // G

_mesh = plsc.VectorSubcoreMesh(
    core_axis_name="c", subcore_axis_name="s", num_cores=NC, num_subcores=NS)


@functools.partial(
    pl.kernel,
    mesh=_mesh,
    out_type=(
        jax.ShapeDtypeStruct((DIM, BATCH), jnp.float32),
        jax.ShapeDtypeStruct((DIM, BATCH), jnp.float32),
    ),
    scratch_types=(
        pltpu.VMEM((B_PER_W,), jnp.int32),
        pltpu.VMEM((B_PER_W,), jnp.int32),
        pltpu.VMEM((B_PER_W,), jnp.int32),
        pltpu.VMEM((G, DIM, 128), jnp.float32),
        pltpu.VMEM((G, DIM, 128), jnp.float32),
        pltpu.VMEM((G, DIM, 128), jnp.float32),
        pltpu.VMEM((DIM, B_PER_W), jnp.float32),
        pltpu.VMEM((DIM, B_PER_W), jnp.float32),
        pltpu.SemaphoreType.DMA,
    ),
    compiler_params=pltpu.CompilerParams(
        use_tc_tiling_on_sc=True, needs_layout_passes=False),
)
def _bdl_fwd(u_hbm, i_hbm, j_hbm, uwT, iwT, out_ui, out_uj,
             idx_u, idx_i, idx_j, gbu, gbi, gbj, obu, obj, sem):
    wid = lax.axis_index("s") * NC + lax.axis_index("c")
    base = wid * B_PER_W
    pltpu.sync_copy(u_hbm.at[pl.ds(base, B_PER_W)], idx_u)
    pltpu.sync_copy(i_hbm.at[pl.ds(base, B_PER_W)], idx_i)
    pltpu.sync_copy(j_hbm.at[pl.ds(base, B_PER_W)], idx_j)

    iota16 = lax.iota(jnp.int32, G)

    def group_body(g, _):
        g16 = g * G
        iv_u = idx_u[pl.ds(g16, G)]
        iv_i = idx_i[pl.ds(g16, G)]
        iv_j = idx_j[pl.ds(g16, G)]
        cv_u = (iv_u >> 7) << 7
        cv_i = (iv_i >> 7) << 7
        cv_j = (iv_j >> 7) << 7
        lv_u = iv_u - cv_u
        lv_i = iv_i - cv_i
        lv_j = iv_j - cv_j

        cps = []
        for (cv, tbl, gb) in ((cv_u, uwT, gbu),
                              (cv_i, iwT, gbi),
                              (cv_j, iwT, gbj)):
            for i in range(G):
                c = pl.multiple_of(cv[i], 128)
                cps.append(pltpu.async_copy(
                    tbl.at[:, pl.ds(c, 128)], gb.at[i], sem))
        for cp in cps:
            cp.wait()

        for f in range(DIM):
            fv = jnp.full((G,), f, jnp.int32)
            uf = plsc.load_gather(gbu, [iota16, fv, lv_u])
            vf = plsc.load_gather(gbi, [iota16, fv, lv_i])
            wf = plsc.load_gather(gbj, [iota16, fv, lv_j])
            obu[f, pl.ds(g16, G)] = uf * vf
            obj[f, pl.ds(g16, G)] = uf * wf
        return 0

    lax.fori_loop(0, NG, group_body, 0)
    pltpu.sync_copy(obu, out_ui.at[:, pl.ds(base, B_PER_W)])
    pltpu.sync_copy(obj, out_uj.at[:, pl.ds(base, B_PER_W)])


def kernel(user, item, neg_item, user_emb_w, item_emb_w):
    h_uiT, h_ujT = _bdl_fwd(user.astype(jnp.int32), item.astype(jnp.int32),
                            neg_item.astype(jnp.int32),
                            user_emb_w.T, item_emb_w.T)
    return (h_uiT.T, h_ujT.T)
